# baseline (speedup 1.0000x reference)
import jax
import jax.numpy as jnp
from jax.experimental import pallas as pl

N = 10000
D_IN = 256
D_OUT = 256

BM1 = 2000
BM2 = 400


def _support_kernel(x_ref, w_ref, s_ref):
    acc = jnp.dot(
        x_ref[...].astype(jnp.bfloat16),
        w_ref[...],
        preferred_element_type=jnp.float32,
    )
    s_ref[...] = acc.astype(jnp.bfloat16)


def _agg_kernel(adj_ref, s_ref, o_ref):
    o_ref[...] = jnp.dot(
        adj_ref[...].astype(jnp.bfloat16),
        s_ref[...],
        preferred_element_type=jnp.float32,
    )


def kernel(input, adj, origin_features, weight, weight2):
    w_bf = weight.astype(jnp.bfloat16)
    support = pl.pallas_call(
        _support_kernel,
        grid=(N // BM1,),
        in_specs=[
            pl.BlockSpec((BM1, D_IN), lambda i: (i, 0)),
            pl.BlockSpec((D_IN, D_OUT), lambda i: (0, 0)),
        ],
        out_specs=pl.BlockSpec((BM1, D_OUT), lambda i: (i, 0)),
        out_shape=jax.ShapeDtypeStruct((N, D_OUT), jnp.bfloat16),
    )(input, w_bf)
    out = pl.pallas_call(
        _agg_kernel,
        grid=(N // BM2,),
        in_specs=[
            pl.BlockSpec((BM2, N), lambda i: (i, 0)),
            pl.BlockSpec((N, D_OUT), lambda i: (0, 0)),
        ],
        out_specs=pl.BlockSpec((BM2, D_OUT), lambda i: (i, 0)),
        out_shape=jax.ShapeDtypeStruct((N, D_OUT), jnp.float32),
    )(adj, support)
    return out


# speedup vs baseline: 1.0324x; 1.0324x over previous
import jax
import jax.numpy as jnp
from jax.experimental import pallas as pl
from jax.experimental.pallas import tpu as pltpu

N = 10000
D_IN = 256
D_OUT = 256

BM = 400
NB = N // BM


def _fused_kernel(x_ref, w_ref, adj_ref, o_ref, s_ref):
    i = pl.program_id(0)

    @pl.when(i == 0)
    def _():
        acc = jnp.dot(
            x_ref[...].astype(jnp.bfloat16),
            w_ref[...],
            preferred_element_type=jnp.float32,
        )
        s_ref[...] = acc.astype(jnp.bfloat16)

    @pl.when(i > 0)
    def _():
        o_ref[...] = jnp.dot(
            adj_ref[...].astype(jnp.bfloat16),
            s_ref[...],
            preferred_element_type=jnp.float32,
        )


def kernel(input, adj, origin_features, weight, weight2):
    w_bf = weight.astype(jnp.bfloat16)
    out = pl.pallas_call(
        _fused_kernel,
        grid=(NB + 1,),
        in_specs=[
            pl.BlockSpec((N, D_IN), lambda i: (0, 0)),
            pl.BlockSpec((D_IN, D_OUT), lambda i: (0, 0)),
            pl.BlockSpec((BM, N), lambda i: (jnp.maximum(i - 1, 0), 0)),
        ],
        out_specs=pl.BlockSpec((BM, D_OUT), lambda i: (jnp.maximum(i - 1, 0), 0)),
        out_shape=jax.ShapeDtypeStruct((N, D_OUT), jnp.float32),
        scratch_shapes=[pltpu.VMEM((N, D_OUT), jnp.bfloat16)],
    )(input, w_bf, adj)
    return out


# speedup vs baseline: 1.0396x; 1.0070x over previous
import jax
import jax.numpy as jnp
from jax.experimental import pallas as pl
from jax.experimental.pallas import tpu as pltpu

N = 10000
D_IN = 256
D_OUT = 256

BM = 200
NB = N // BM


def _fused_kernel(x_ref, w_ref, adj_ref, o_ref, s_ref):
    i = pl.program_id(0)

    @pl.when(i == 0)
    def _():
        acc = jnp.dot(
            x_ref[...].astype(jnp.bfloat16),
            w_ref[...],
            preferred_element_type=jnp.float32,
        )
        s_ref[...] = acc.astype(jnp.bfloat16)

    o_ref[...] = jnp.dot(
        adj_ref[...].astype(jnp.bfloat16),
        s_ref[...],
        preferred_element_type=jnp.float32,
    )


def kernel(input, adj, origin_features, weight, weight2):
    w_bf = weight.astype(jnp.bfloat16)
    out = pl.pallas_call(
        _fused_kernel,
        grid=(NB,),
        in_specs=[
            pl.BlockSpec((N, D_IN), lambda i: (0, 0)),
            pl.BlockSpec((D_IN, D_OUT), lambda i: (0, 0)),
            pl.BlockSpec((BM, N), lambda i: (i, 0)),
        ],
        out_specs=pl.BlockSpec((BM, D_OUT), lambda i: (i, 0)),
        out_shape=jax.ShapeDtypeStruct((N, D_OUT), jnp.float32),
        scratch_shapes=[pltpu.VMEM((N, D_OUT), jnp.bfloat16)],
    )(input, w_bf, adj)
    return out


# speedup vs baseline: 1.0412x; 1.0015x over previous
import jax
import jax.numpy as jnp
from jax.experimental import pallas as pl
from jax.experimental.pallas import tpu as pltpu

N = 10000
D_IN = 256
D_OUT = 256

BM = 200
NB = N // BM
NBUF = 4


def _adj_copy(adj_hbm, adj_buf, sems, blk, slot):
    return pltpu.make_async_copy(
        adj_hbm.at[pl.ds(blk * BM, BM), :],
        adj_buf.at[slot],
        sems.at[slot],
    )


def _fused_kernel(x_ref, w_ref, adj_hbm, o_ref, s_ref, adj_buf, sems):
    i = pl.program_id(0)

    @pl.when(i == 0)
    def _():
        for b in range(NBUF - 1):
            _adj_copy(adj_hbm, adj_buf, sems, b, b).start()
        acc = jnp.dot(
            x_ref[...].astype(jnp.bfloat16),
            w_ref[...],
            preferred_element_type=jnp.float32,
        )
        s_ref[...] = acc.astype(jnp.bfloat16)

    nxt = i + NBUF - 1

    @pl.when(nxt < NB)
    def _():
        _adj_copy(adj_hbm, adj_buf, sems, nxt, jax.lax.rem(nxt, NBUF)).start()

    slot = jax.lax.rem(i, NBUF)
    _adj_copy(adj_hbm, adj_buf, sems, i, slot).wait()
    o_ref[...] = jnp.dot(
        adj_buf[slot].astype(jnp.bfloat16),
        s_ref[...],
        preferred_element_type=jnp.float32,
    )


def kernel(input, adj, origin_features, weight, weight2):
    w_bf = weight.astype(jnp.bfloat16)
    out = pl.pallas_call(
        _fused_kernel,
        grid=(NB,),
        in_specs=[
            pl.BlockSpec((N, D_IN), lambda i: (0, 0)),
            pl.BlockSpec((D_IN, D_OUT), lambda i: (0, 0)),
            pl.BlockSpec(memory_space=pl.ANY),
        ],
        out_specs=pl.BlockSpec((BM, D_OUT), lambda i: (i, 0)),
        out_shape=jax.ShapeDtypeStruct((N, D_OUT), jnp.float32),
        scratch_shapes=[
            pltpu.VMEM((N, D_OUT), jnp.bfloat16),
            pltpu.VMEM((NBUF, BM, N), jnp.float32),
            pltpu.SemaphoreType.DMA((NBUF,)),
        ],
    )(input, w_bf, adj)
    return out
